# Initial kernel scaffold; baseline (speedup 1.0000x reference)
import jax
import jax.numpy as jnp
from jax.experimental import pallas as pl


def kernel(center, feat, affine_alpha, affine_beta, ln_gamma, ln_beta, conv_w, conv_b):
    raise NotImplementedError("write your pallas kernel here")


# speedup vs baseline: 12.0105x; 12.0105x over previous
import functools

import jax
import jax.numpy as jnp
from jax.experimental import pallas as pl

_K = 16
_G = 512
_C = 384


def _phase1_body(center_ref, centerT_ref, x_ref, p_ref, part_ref):
    c3 = center_ref[0]
    cT = centerT_ref[0]
    x = x_ref[0]
    sq = jnp.sum(c3 * c3, axis=1, keepdims=True)
    sqT = jnp.sum(cT * cT, axis=0, keepdims=True)
    dot = jnp.dot(c3, cT, preferred_element_type=jnp.float32)
    d = sq + sqT - 2.0 * dot

    iota_h = jax.lax.broadcasted_iota(jnp.int32, (_G, _G), 1)
    big = jnp.float32(3e38)

    def step(_, carry):
        key, P = carry
        m = jnp.min(key, axis=1, keepdims=True)
        eq = key == m
        hsel = jnp.min(jnp.where(eq, iota_h, jnp.int32(1 << 30)),
                       axis=1, keepdims=True)
        onehot = iota_h == hsel
        P = P + onehot.astype(jnp.float32)
        key = jnp.where(onehot, big, key)
        return key, P

    _, P = jax.lax.fori_loop(
        0, _K, step, (d, jnp.zeros((_G, _G), jnp.float32)))
    p_ref[0] = P

    xcat = jnp.concatenate([x, x * x], axis=1)
    S = jnp.dot(P, xcat, preferred_element_type=jnp.float32)
    SX = S[:, :_C]
    SX2 = S[:, _C:]
    kf = jnp.float32(_K)
    s1 = jnp.sum(SX) - kf * jnp.sum(x)
    s2 = jnp.sum(SX2) - 2.0 * jnp.sum(x * SX) + kf * jnp.sum(x * x)
    lane = jax.lax.broadcasted_iota(jnp.int32, (1, 128), 1)
    row = jnp.where(lane == 0, s1, jnp.where(lane == 1, s2, 0.0))
    part_ref[0] = row


def _phase2_body(p_ref, x_ref, part_ref, aa_ref, ab_ref, lg_ref, lb_ref,
                 cwT_ref, cb_ref, out_ref):
    P = p_ref[0]
    x = x_ref[0]
    parts = part_ref[...]
    lane = jax.lax.broadcasted_iota(jnp.int32, parts.shape, 2)
    s1 = jnp.sum(jnp.where(lane == 0, parts, 0.0))
    s2 = jnp.sum(jnp.where(lane == 1, parts, 0.0))
    n = jnp.float32(parts.shape[0] * _G * _K * _C)
    m = s1 / n
    var = (s2 - n * m * m) / (n - 1.0)
    sp = jnp.sqrt(var) + jnp.float32(1e-5)

    alpha = aa_ref[...]
    beta = ab_ref[...]
    a1 = alpha[:, :_C] / sp
    b1 = beta[:, :_C]
    a2 = alpha[:, _C:]
    b2 = beta[:, _C:]

    ax = x * a1
    E = jnp.exp(ax)
    F = ax * E
    EF = jnp.concatenate([E, F], axis=1)
    S = jnp.dot(P, EF, preferred_element_type=jnp.float32)
    SE = S[:, :_C]
    SF = S[:, _C:]
    first = SF / SE + b1 - ax
    second = x * a2 + b2
    y = jnp.concatenate([first, second], axis=1)

    mu = jnp.mean(y, axis=1, keepdims=True)
    v2 = jnp.mean((y - mu) ** 2, axis=1, keepdims=True)
    y = (y - mu) / jnp.sqrt(v2 + 1e-5) * lg_ref[...] + lb_ref[...]

    out = jnp.dot(y, cwT_ref[...], preferred_element_type=jnp.float32)
    out = out + cb_ref[...]
    out = out / (1.0 + jnp.exp(-out))
    out_ref[0] = out


def kernel(center, feat, affine_alpha, affine_beta, ln_gamma, ln_beta,
           conv_w, conv_b):
    B, Gp1, C = feat.shape
    G = Gp1 - 1
    C2 = 2 * C
    cls_token = feat[:, :1, :]
    x = feat[:, 1:, :]
    centerT = jnp.transpose(center, (0, 2, 1))
    aa = affine_alpha.reshape(1, C2)
    ab = affine_beta.reshape(1, C2)
    lg = ln_gamma.reshape(1, C2)
    lb = ln_beta.reshape(1, C2)
    cwT = conv_w.T
    cb = conv_b.reshape(1, C)

    P, parts = pl.pallas_call(
        _phase1_body,
        grid=(B,),
        in_specs=[
            pl.BlockSpec((1, G, 3), lambda b: (b, 0, 0)),
            pl.BlockSpec((1, 3, G), lambda b: (b, 0, 0)),
            pl.BlockSpec((1, G, C), lambda b: (b, 0, 0)),
        ],
        out_specs=[
            pl.BlockSpec((1, G, G), lambda b: (b, 0, 0)),
            pl.BlockSpec((1, 1, 128), lambda b: (b, 0, 0)),
        ],
        out_shape=[
            jax.ShapeDtypeStruct((B, G, G), jnp.float32),
            jax.ShapeDtypeStruct((B, 1, 128), jnp.float32),
        ],
    )(center, centerT, x)

    lc = pl.pallas_call(
        _phase2_body,
        grid=(B,),
        in_specs=[
            pl.BlockSpec((1, G, G), lambda b: (b, 0, 0)),
            pl.BlockSpec((1, G, C), lambda b: (b, 0, 0)),
            pl.BlockSpec((B, 1, 128), lambda b: (0, 0, 0)),
            pl.BlockSpec((1, C2), lambda b: (0, 0)),
            pl.BlockSpec((1, C2), lambda b: (0, 0)),
            pl.BlockSpec((1, C2), lambda b: (0, 0)),
            pl.BlockSpec((1, C2), lambda b: (0, 0)),
            pl.BlockSpec((C2, C), lambda b: (0, 0)),
            pl.BlockSpec((1, C), lambda b: (0, 0)),
        ],
        out_specs=pl.BlockSpec((1, G, C), lambda b: (b, 0, 0)),
        out_shape=jax.ShapeDtypeStruct((B, G, C), jnp.float32),
    )(P, x, parts, aa, ab, lg, lb, cwT, cb)

    return jnp.concatenate([cls_token, lc], axis=1)
